# baseline (speedup 1.0000x reference)
import jax
import jax.numpy as jnp
import numpy as np
from jax.experimental import pallas as pl
from jax.experimental.pallas import tpu as pltpu

PATCH_L = 512
PATCH_C = 64
NUM_PATCHES = 16
F_S = 100.0

_N = PATCH_L
_kn = (np.arange(_N)[:, None] * np.arange(_N)[None, :]) % _N
_ang = 2.0 * np.pi * _kn / _N
_DFT = np.concatenate([np.cos(_ang), -np.sin(_ang)], axis=0).astype(np.float32)


def _patch_starts(B, L, C):
    kL, kC = jax.random.split(jax.random.key(42))
    start_L = jax.random.randint(kL, (B, NUM_PATCHES), 0, L - PATCH_L + 1)
    start_C = jax.random.randint(kC, (B, NUM_PATCHES), 0, C - PATCH_C + 1)
    return start_L, start_C


def _fft_body(sl_ref, sc_ref, x_ref, dft_ref, ore_ref, oim_ref):
    b = pl.program_id(0)
    C = x_ref.shape[2]
    cols = []
    for p in range(NUM_PATCHES):
        i = b * NUM_PATCHES + p
        sl = sl_ref[i]
        sc = sc_ref[i]
        xs = x_ref[0, pl.ds(sl, PATCH_L), :]
        cols.append(pltpu.roll(xs, C - sc, axis=1)[:, :PATCH_C])
    patches = jnp.concatenate(cols, axis=1)
    res = jax.lax.dot_general(
        dft_ref[...], patches, (((1,), (0,)), ((), ())),
        preferred_element_type=jnp.float32,
    )
    for p in range(NUM_PATCHES):
        ore_ref[0, p] = res[:PATCH_L, p * PATCH_C:(p + 1) * PATCH_C]
        oim_ref[0, p] = res[PATCH_L:, p * PATCH_C:(p + 1) * PATCH_C]


def kernel(x):
    B, L, C = x.shape
    start_L, start_C = _patch_starts(B, L, C)
    sl_flat = start_L.reshape(-1).astype(jnp.int32)
    sc_flat = start_C.reshape(-1).astype(jnp.int32)
    dft = jnp.asarray(_DFT)

    grid_spec = pltpu.PrefetchScalarGridSpec(
        num_scalar_prefetch=2,
        grid=(B,),
        in_specs=[
            pl.BlockSpec((1, L, C), lambda b, *_: (b, 0, 0)),
            pl.BlockSpec((2 * PATCH_L, PATCH_L), lambda b, *_: (0, 0)),
        ],
        out_specs=[
            pl.BlockSpec((1, NUM_PATCHES, PATCH_L, PATCH_C),
                         lambda b, *_: (b, 0, 0, 0)),
            pl.BlockSpec((1, NUM_PATCHES, PATCH_L, PATCH_C),
                         lambda b, *_: (b, 0, 0, 0)),
        ],
    )
    ore, oim = pl.pallas_call(
        _fft_body,
        grid_spec=grid_spec,
        out_shape=[
            jax.ShapeDtypeStruct((B, NUM_PATCHES, PATCH_L, PATCH_C), jnp.float32),
            jax.ShapeDtypeStruct((B, NUM_PATCHES, PATCH_L, PATCH_C), jnp.float32),
        ],
    )(sl_flat, sc_flat, x, dft)

    patches_fft = jnp.stack([ore, oim], axis=-1)
    t = jnp.broadcast_to(
        (jnp.arange(L, dtype=jnp.float32) * (1.0 / F_S))[None, :], (B, L)
    )
    return (patches_fft, t)
